# baseline (speedup 1.0000x reference)
import functools

import jax
import jax.numpy as jnp
from jax import lax
from jax.experimental import pallas as pl
from jax.experimental.pallas import tpu as pltpu
from jax.experimental.pallas import tpu_sc as plsc

NC = 2
NS = 16
LANES = 16
CHUNK = 128
NBUF = 2


def _sc_degree(dst2d, n_acc, e_pad):
    chunks = e_pad // (NC * NS * CHUNK)
    rpt = n_acc // NS
    mesh = plsc.VectorSubcoreMesh(core_axis_name="c", subcore_axis_name="s")

    zeros16 = jnp.zeros((n_acc, LANES), jnp.float32)
    ones16 = jnp.ones((CHUNK, LANES), jnp.float32)

    @functools.partial(
        pl.kernel,
        out_type=jax.ShapeDtypeStruct((NC * n_acc, LANES), jnp.float32),
        mesh=mesh,
        scratch_types=[
            pltpu.VMEM((chunks, CHUNK), jnp.int32),
            pltpu.VMEM((CHUNK, LANES), jnp.float32),
            pltpu.VMEM_SHARED((n_acc, LANES), jnp.float32),
        ],
        compiler_params=pltpu.CompilerParams(use_tc_tiling_on_sc=False),
    )
    def deg_kernel(dst_hbm, z_hbm, ones_hbm, out_hbm, didx_v, ones_v, acc_sh):
        cid = lax.axis_index("c")
        sid = lax.axis_index("s")
        crow = (cid * NS + sid) * chunks
        pltpu.sync_copy(ones_hbm, ones_v)
        pltpu.sync_copy(dst_hbm.at[pl.ds(crow, chunks)], didx_v)
        pltpu.sync_copy(z_hbm.at[pl.ds(sid * rpt, rpt)],
                        acc_sh.at[pl.ds(sid * rpt, rpt)])
        plsc.subcore_barrier()

        def body(i, carry):
            pltpu.sync_copy(ones_v, acc_sh.at[didx_v.at[i]], add=True)
            return carry

        lax.fori_loop(0, chunks, body, 0)
        plsc.subcore_barrier()
        pltpu.sync_copy(acc_sh.at[pl.ds(sid * rpt, rpt)],
                        out_hbm.at[pl.ds(cid * n_acc + sid * rpt, rpt)])

    return deg_kernel(dst2d, zeros16, ones16)


def _sc_message(g, src2d, dst2d, n_acc, e_pad):
    d = g.shape[1]
    chunks = e_pad // (NC * NS * CHUNK)
    rpt = n_acc // NS
    groups = chunks // NBUF
    mesh = plsc.VectorSubcoreMesh(core_axis_name="c", subcore_axis_name="s")

    zeros = jnp.zeros((n_acc, d), jnp.float32)

    @functools.partial(
        pl.kernel,
        out_type=jax.ShapeDtypeStruct((NC * n_acc, d), jnp.float32),
        mesh=mesh,
        scratch_types=[pltpu.VMEM((chunks, CHUNK), jnp.int32)]
        + [pltpu.VMEM((CHUNK,), jnp.int32) for _ in range(NBUF)]
        + [pltpu.VMEM((CHUNK, d), jnp.float32) for _ in range(NBUF)]
        + [pltpu.VMEM_SHARED((n_acc, d), jnp.float32)]
        + [pltpu.SemaphoreType.DMA for _ in range(2 * NBUF)],
    )
    def msg_kernel(g_hbm, src_hbm, dst_hbm, z_hbm, out_hbm, sidx_v, *rest):
        didx = rest[:NBUF]
        rows = rest[NBUF:2 * NBUF]
        acc_sh = rest[2 * NBUF]
        rsem = rest[2 * NBUF + 1:2 * NBUF + 1 + NBUF]
        dsem = rest[2 * NBUF + 1 + NBUF:]
        cid = lax.axis_index("c")
        sid = lax.axis_index("s")
        crow = (cid * NS + sid) * chunks
        pltpu.sync_copy(src_hbm.at[pl.ds(crow, chunks)], sidx_v)
        pltpu.sync_copy(z_hbm.at[pl.ds(sid * rpt, rpt)],
                        acc_sh.at[pl.ds(sid * rpt, rpt)])
        plsc.subcore_barrier()

        for b in range(NBUF):
            pltpu.async_copy(g_hbm.at[sidx_v.at[b]], rows[b], rsem[b])
            pltpu.async_copy(dst_hbm.at[crow + b], didx[b], dsem[b])

        def body(gidx, carry):
            for b in range(NBUF):
                i = gidx * NBUF + b
                pltpu.make_async_copy(g_hbm.at[pl.ds(0, CHUNK)],
                                      rows[b], rsem[b]).wait()
                pltpu.make_async_copy(dst_hbm.at[0], didx[b], dsem[b]).wait()
                pltpu.sync_copy(rows[b], acc_sh.at[didx[b]], add=True)
                nxt = jnp.minimum(i + NBUF, chunks - 1)
                pltpu.async_copy(g_hbm.at[sidx_v.at[nxt]], rows[b], rsem[b])
                pltpu.async_copy(dst_hbm.at[crow + nxt], didx[b], dsem[b])
            return carry

        lax.fori_loop(0, groups, body, 0)
        for b in range(NBUF):
            pltpu.make_async_copy(g_hbm.at[pl.ds(0, CHUNK)],
                                  rows[b], rsem[b]).wait()
            pltpu.make_async_copy(dst_hbm.at[0], didx[b], dsem[b]).wait()
        plsc.subcore_barrier()
        pltpu.sync_copy(acc_sh.at[pl.ds(sid * rpt, rpt)],
                        out_hbm.at[pl.ds(cid * n_acc + sid * rpt, rpt)])

    return msg_kernel(g, src2d, dst2d, zeros)


def _dinv_from_parts(degp_ref, n):
    deg = degp_ref[0] + degp_ref[1]
    return lax.rsqrt(deg[:n, 0:1] + 1.0)


def _tc_in(x, w1, degp, n):
    def body(x_ref, w_ref, degp_ref, g_ref):
        dinv = _dinv_from_parts(degp_ref, n)
        h = jnp.dot(x_ref[...], w_ref[...], preferred_element_type=jnp.float32)
        g_ref[...] = h * dinv

    return pl.pallas_call(
        body, out_shape=jax.ShapeDtypeStruct((n, x.shape[1]), jnp.float32),
    )(x, w1, degp)


def _tc_mid(parts, gprev, degp, b, w, n):
    def body(p_ref, g_ref, degp_ref, b_ref, w_ref, o_ref):
        dinv = _dinv_from_parts(degp_ref, n)
        s = p_ref[0, :n, :] + p_ref[1, :n, :] + g_ref[...]
        h = jnp.maximum(dinv * s + b_ref[...], 0.0)
        o_ref[...] = dinv * jnp.dot(h, w_ref[...],
                                    preferred_element_type=jnp.float32)

    return pl.pallas_call(
        body, out_shape=jax.ShapeDtypeStruct((n, w.shape[1]), jnp.float32),
    )(parts, gprev, degp, b, w)


def _tc_out(parts, gprev, degp, b, w, bfc, n):
    def body(p_ref, g_ref, degp_ref, b_ref, w_ref, bfc_ref, o_ref):
        dinv = _dinv_from_parts(degp_ref, n)
        s = p_ref[0, :n, :] + p_ref[1, :n, :] + g_ref[...]
        h = jnp.maximum(dinv * s + b_ref[...], 0.0)
        o_ref[...] = jnp.dot(h, w_ref[...],
                             preferred_element_type=jnp.float32) + bfc_ref[...]

    return pl.pallas_call(
        body, out_shape=jax.ShapeDtypeStruct((n, w.shape[1]), jnp.float32),
    )(parts, gprev, degp, b, w, bfc)


def kernel(x, edge_index, W1, b1, W2, b2, Wfc, bfc):
    n, d_in = x.shape
    e = edge_index.shape[1]

    n_acc = -(-(n + 1) // (NS * 8)) * (NS * 8)
    grain = NC * NS * CHUNK * NBUF
    e_pad = -(-e // grain) * grain

    src = edge_index[0].astype(jnp.int32)
    dst = edge_index[1].astype(jnp.int32)
    src2d = jnp.concatenate(
        [src, jnp.zeros((e_pad - e,), jnp.int32)]).reshape(-1, CHUNK)
    dst2d = jnp.concatenate(
        [dst, jnp.full((e_pad - e,), n, jnp.int32)]).reshape(-1, CHUNK)

    degp = _sc_degree(dst2d, n_acc, e_pad).reshape(NC, n_acc, LANES)

    g1 = _tc_in(x, W1, degp, n)
    p1 = _sc_message(g1, src2d, dst2d, n_acc, e_pad).reshape(NC, n_acc, d_in)
    g2 = _tc_mid(p1, g1, degp, b1.reshape(1, -1), W2, n)
    p2 = _sc_message(g2, src2d, dst2d, n_acc, e_pad).reshape(NC, n_acc, d_in)
    out = _tc_out(p2, g2, degp, b2.reshape(1, -1), Wfc, bfc.reshape(1, -1), n)
    return out


# speedup vs baseline: 1.6960x; 1.6960x over previous
import functools

import jax
import jax.numpy as jnp
from jax import lax
from jax.experimental import pallas as pl
from jax.experimental.pallas import tpu as pltpu
from jax.experimental.pallas import tpu_sc as plsc

NC = 2
NS = 16
LANES = 16
CHUNK = 128
HALF = 64


def _sc_degree(dst2d, n_acc, e_pad):
    chunks = e_pad // (NC * NS * CHUNK)
    rpt = n_acc // NS
    mesh = plsc.VectorSubcoreMesh(core_axis_name="c", subcore_axis_name="s")

    zeros16 = jnp.zeros((n_acc, LANES), jnp.float32)
    ones16 = jnp.ones((CHUNK, LANES), jnp.float32)

    @functools.partial(
        pl.kernel,
        out_type=jax.ShapeDtypeStruct((NC * n_acc, LANES), jnp.float32),
        mesh=mesh,
        scratch_types=[
            pltpu.VMEM((chunks, CHUNK), jnp.int32),
            pltpu.VMEM((CHUNK, LANES), jnp.float32),
            pltpu.VMEM_SHARED((n_acc, LANES), jnp.float32),
        ],
        compiler_params=pltpu.CompilerParams(use_tc_tiling_on_sc=False),
    )
    def deg_kernel(dst_hbm, z_hbm, ones_hbm, out_hbm, didx_v, ones_v, acc_sh):
        cid = lax.axis_index("c")
        sid = lax.axis_index("s")
        crow = (cid * NS + sid) * chunks
        pltpu.sync_copy(ones_hbm, ones_v)
        pltpu.sync_copy(dst_hbm.at[pl.ds(crow, chunks)], didx_v)
        pltpu.sync_copy(z_hbm.at[pl.ds(sid * rpt, rpt)],
                        acc_sh.at[pl.ds(sid * rpt, rpt)])
        plsc.subcore_barrier()

        def body(i, carry):
            pltpu.sync_copy(ones_v, acc_sh.at[didx_v.at[i]], add=True)
            return carry

        lax.fori_loop(0, chunks, body, 0)
        plsc.subcore_barrier()
        pltpu.sync_copy(acc_sh.at[pl.ds(sid * rpt, rpt)],
                        out_hbm.at[pl.ds(cid * n_acc + sid * rpt, rpt)])

    return deg_kernel(dst2d, zeros16, ones16)


def _sc_message(gA, gB, src2d, dst2d, n_acc, e_pad):
    chunks = e_pad // (NC * NS * CHUNK)
    rpt = n_acc // NS
    mesh = plsc.VectorSubcoreMesh(core_axis_name="c", subcore_axis_name="s")

    zeros = jnp.zeros((n_acc, HALF), jnp.float32)
    out_sds = jax.ShapeDtypeStruct((NC * n_acc, HALF), jnp.float32)

    @functools.partial(
        pl.kernel,
        out_type=(out_sds, out_sds),
        mesh=mesh,
        scratch_types=[
            pltpu.VMEM((chunks, CHUNK), jnp.int32),
            pltpu.VMEM((chunks, CHUNK), jnp.int32),
            pltpu.VMEM((CHUNK, HALF), jnp.float32),
            pltpu.VMEM_SHARED((n_acc, HALF), jnp.float32),
            pltpu.VMEM_SHARED((n_acc, HALF), jnp.float32),
            pltpu.SemaphoreType.DMA,
        ],
        compiler_params=pltpu.CompilerParams(use_tc_tiling_on_sc=False),
    )
    def msg_kernel(gA_hbm, gB_hbm, src_hbm, dst_hbm, z_hbm, outA, outB,
                   sidx_v, didx_v, rows_v, g_sh, acc_sh, sem):
        cid = lax.axis_index("c")
        sid = lax.axis_index("s")
        crow = (cid * NS + sid) * chunks
        pltpu.sync_copy(src_hbm.at[pl.ds(crow, chunks)], sidx_v)
        pltpu.sync_copy(dst_hbm.at[pl.ds(crow, chunks)], didx_v)

        for g_hbm, out_hbm in ((gA_hbm, outA), (gB_hbm, outB)):
            pltpu.sync_copy(g_hbm.at[pl.ds(sid * rpt, rpt)],
                            g_sh.at[pl.ds(sid * rpt, rpt)])
            pltpu.sync_copy(z_hbm.at[pl.ds(sid * rpt, rpt)],
                            acc_sh.at[pl.ds(sid * rpt, rpt)])
            plsc.subcore_barrier()

            def body(i, carry):
                pltpu.async_copy(g_sh.at[sidx_v.at[i]], rows_v, sem).wait()
                pltpu.sync_copy(rows_v, acc_sh.at[didx_v.at[i]], add=True)
                return carry

            lax.fori_loop(0, chunks, body, 0)
            plsc.subcore_barrier()
            pltpu.sync_copy(acc_sh.at[pl.ds(sid * rpt, rpt)],
                            out_hbm.at[pl.ds(cid * n_acc + sid * rpt, rpt)])

    return msg_kernel(gA, gB, src2d, dst2d, zeros)


def _dinv_from_parts(degp_ref, n):
    deg = degp_ref[0] + degp_ref[1]
    return lax.rsqrt(deg[:n, 0:1] + 1.0)


def _tc_in(x, w1, degp, n, n_acc):
    def body(x_ref, w_ref, degp_ref, gA_ref, gB_ref):
        dinv = _dinv_from_parts(degp_ref, n)
        h = jnp.dot(x_ref[...], w_ref[...], preferred_element_type=jnp.float32)
        g = h * dinv
        gA_ref[:n, :] = g[:, :HALF]
        gB_ref[:n, :] = g[:, HALF:]

    sds = jax.ShapeDtypeStruct((n_acc, HALF), jnp.float32)
    return pl.pallas_call(body, out_shape=(sds, sds))(x, w1, degp)


def _tc_mid(pA, pB, gA, gB, degp, b, w, n, n_acc):
    def body(pA_ref, pB_ref, gA_ref, gB_ref, degp_ref, b_ref, w_ref,
             oA_ref, oB_ref):
        dinv = _dinv_from_parts(degp_ref, n)
        sA = pA_ref[0, :n, :] + pA_ref[1, :n, :] + gA_ref[:n, :]
        sB = pB_ref[0, :n, :] + pB_ref[1, :n, :] + gB_ref[:n, :]
        s = jnp.concatenate([sA, sB], axis=1)
        h = jnp.maximum(dinv * s + b_ref[...], 0.0)
        g = dinv * jnp.dot(h, w_ref[...], preferred_element_type=jnp.float32)
        oA_ref[:n, :] = g[:, :HALF]
        oB_ref[:n, :] = g[:, HALF:]

    sds = jax.ShapeDtypeStruct((n_acc, HALF), jnp.float32)
    return pl.pallas_call(body, out_shape=(sds, sds))(
        pA, pB, gA, gB, degp, b, w)


def _tc_out(pA, pB, gA, gB, degp, b, w, bfc, n):
    def body(pA_ref, pB_ref, gA_ref, gB_ref, degp_ref, b_ref, w_ref,
             bfc_ref, o_ref):
        dinv = _dinv_from_parts(degp_ref, n)
        sA = pA_ref[0, :n, :] + pA_ref[1, :n, :] + gA_ref[:n, :]
        sB = pB_ref[0, :n, :] + pB_ref[1, :n, :] + gB_ref[:n, :]
        s = jnp.concatenate([sA, sB], axis=1)
        h = jnp.maximum(dinv * s + b_ref[...], 0.0)
        o_ref[...] = jnp.dot(h, w_ref[...],
                             preferred_element_type=jnp.float32) + bfc_ref[...]

    return pl.pallas_call(
        body, out_shape=jax.ShapeDtypeStruct((n, w.shape[1]), jnp.float32),
    )(pA, pB, gA, gB, degp, b, w, bfc)


def kernel(x, edge_index, W1, b1, W2, b2, Wfc, bfc):
    n, d_in = x.shape
    e = edge_index.shape[1]

    n_acc = -(-(n + 1) // (NS * 8)) * (NS * 8)
    grain = NC * NS * CHUNK
    e_pad = -(-e // grain) * grain

    src = edge_index[0].astype(jnp.int32)
    dst = edge_index[1].astype(jnp.int32)
    src2d = jnp.concatenate(
        [src, jnp.zeros((e_pad - e,), jnp.int32)]).reshape(-1, CHUNK)
    dst2d = jnp.concatenate(
        [dst, jnp.full((e_pad - e,), n, jnp.int32)]).reshape(-1, CHUNK)

    degp = _sc_degree(dst2d, n_acc, e_pad).reshape(NC, n_acc, LANES)

    g1A, g1B = _tc_in(x, W1, degp, n, n_acc)
    p1A, p1B = _sc_message(g1A, g1B, src2d, dst2d, n_acc, e_pad)
    g2A, g2B = _tc_mid(p1A.reshape(NC, n_acc, HALF), p1B.reshape(NC, n_acc, HALF),
                       g1A, g1B, degp, b1.reshape(1, -1), W2, n, n_acc)
    p2A, p2B = _sc_message(g2A, g2B, src2d, dst2d, n_acc, e_pad)
    out = _tc_out(p2A.reshape(NC, n_acc, HALF), p2B.reshape(NC, n_acc, HALF),
                  g2A, g2B, degp, b2.reshape(1, -1), Wfc, bfc.reshape(1, -1), n)
    return out


# speedup vs baseline: 2.1084x; 1.2432x over previous
import functools

import jax
import jax.numpy as jnp
from jax import lax
from jax.experimental import pallas as pl
from jax.experimental.pallas import tpu as pltpu
from jax.experimental.pallas import tpu_sc as plsc

NC = 2
NS = 16
LANES = 16
CHUNK = 128
HALF = 64
NBUF = 2


def _sc_degree(dst2d, n_acc, e_pad):
    chunks = e_pad // (NC * NS * CHUNK)
    rpt = n_acc // NS
    mesh = plsc.VectorSubcoreMesh(core_axis_name="c", subcore_axis_name="s")

    zeros16 = jnp.zeros((n_acc, LANES), jnp.float32)
    ones16 = jnp.ones((CHUNK, LANES), jnp.float32)

    @functools.partial(
        pl.kernel,
        out_type=jax.ShapeDtypeStruct((NC * n_acc, LANES), jnp.float32),
        mesh=mesh,
        scratch_types=[
            pltpu.VMEM((chunks, CHUNK), jnp.int32),
            pltpu.VMEM((CHUNK, LANES), jnp.float32),
            pltpu.VMEM_SHARED((n_acc, LANES), jnp.float32),
        ],
        compiler_params=pltpu.CompilerParams(use_tc_tiling_on_sc=False),
    )
    def deg_kernel(dst_hbm, z_hbm, ones_hbm, out_hbm, didx_v, ones_v, acc_sh):
        cid = lax.axis_index("c")
        sid = lax.axis_index("s")
        crow = (cid * NS + sid) * chunks
        pltpu.sync_copy(ones_hbm, ones_v)
        pltpu.sync_copy(dst_hbm.at[pl.ds(crow, chunks)], didx_v)
        pltpu.sync_copy(z_hbm.at[pl.ds(sid * rpt, rpt)],
                        acc_sh.at[pl.ds(sid * rpt, rpt)])
        plsc.subcore_barrier()

        def body(i, carry):
            pltpu.sync_copy(ones_v, acc_sh.at[didx_v.at[i]], add=True)
            return carry

        lax.fori_loop(0, chunks, body, 0)
        plsc.subcore_barrier()
        pltpu.sync_copy(acc_sh.at[pl.ds(sid * rpt, rpt)],
                        out_hbm.at[pl.ds(cid * n_acc + sid * rpt, rpt)])

    return deg_kernel(dst2d, zeros16, ones16)


def _sc_message(gA, gB, src2d, dst2d, n_acc, e_pad):
    chunks = e_pad // (NC * NS * CHUNK)
    rpt = n_acc // NS
    groups = chunks // NBUF
    mesh = plsc.VectorSubcoreMesh(core_axis_name="c", subcore_axis_name="s")

    zeros = jnp.zeros((n_acc, HALF), jnp.float32)
    out_sds = jax.ShapeDtypeStruct((NC * n_acc, HALF), jnp.float32)

    @functools.partial(
        pl.kernel,
        out_type=(out_sds, out_sds),
        mesh=mesh,
        scratch_types=[
            pltpu.VMEM((chunks, CHUNK), jnp.int32),
            pltpu.VMEM((chunks, CHUNK), jnp.int32),
        ]
        + [pltpu.VMEM((CHUNK, HALF), jnp.float32) for _ in range(NBUF)]
        + [pltpu.VMEM_SHARED((n_acc, HALF), jnp.float32)] * 2
        + [pltpu.SemaphoreType.DMA for _ in range(NBUF)],
        compiler_params=pltpu.CompilerParams(use_tc_tiling_on_sc=False),
    )
    def msg_kernel(gA_hbm, gB_hbm, src_hbm, dst_hbm, z_hbm, outA, outB,
                   sidx_v, didx_v, *rest):
        rows = rest[:NBUF]
        g_sh, acc_sh = rest[NBUF], rest[NBUF + 1]
        sems = rest[NBUF + 2:]
        cid = lax.axis_index("c")
        sid = lax.axis_index("s")
        crow = (cid * NS + sid) * chunks
        pltpu.sync_copy(src_hbm.at[pl.ds(crow, chunks)], sidx_v)
        pltpu.sync_copy(dst_hbm.at[pl.ds(crow, chunks)], didx_v)

        for g_hbm, out_hbm in ((gA_hbm, outA), (gB_hbm, outB)):
            pltpu.sync_copy(g_hbm.at[pl.ds(sid * rpt, rpt)],
                            g_sh.at[pl.ds(sid * rpt, rpt)])
            pltpu.sync_copy(z_hbm.at[pl.ds(sid * rpt, rpt)],
                            acc_sh.at[pl.ds(sid * rpt, rpt)])
            plsc.subcore_barrier()

            for b in range(NBUF):
                pltpu.async_copy(g_sh.at[sidx_v.at[b]], rows[b], sems[b])

            def body(gidx, carry):
                for b in range(NBUF):
                    i = gidx * NBUF + b
                    pltpu.make_async_copy(g_sh.at[pl.ds(0, CHUNK)],
                                          rows[b], sems[b]).wait()
                    pltpu.sync_copy(rows[b], acc_sh.at[didx_v.at[i]], add=True)
                    nxt = jnp.minimum(i + NBUF, chunks - 1)
                    pltpu.async_copy(g_sh.at[sidx_v.at[nxt]], rows[b], sems[b])
                return carry

            lax.fori_loop(0, groups, body, 0)
            for b in range(NBUF):
                pltpu.make_async_copy(g_sh.at[pl.ds(0, CHUNK)],
                                      rows[b], sems[b]).wait()
            plsc.subcore_barrier()
            pltpu.sync_copy(acc_sh.at[pl.ds(sid * rpt, rpt)],
                            out_hbm.at[pl.ds(cid * n_acc + sid * rpt, rpt)])

    return msg_kernel(gA, gB, src2d, dst2d, zeros)


def _dinv_from_parts(degp_ref, n):
    deg = degp_ref[0] + degp_ref[1]
    return lax.rsqrt(deg[:n, 0:1] + 1.0)


def _tc_in(x, w1, degp, n, n_acc):
    def body(x_ref, w_ref, degp_ref, gA_ref, gB_ref):
        dinv = _dinv_from_parts(degp_ref, n)
        h = jnp.dot(x_ref[...], w_ref[...], preferred_element_type=jnp.float32)
        g = h * dinv
        gA_ref[:n, :] = g[:, :HALF]
        gB_ref[:n, :] = g[:, HALF:]

    sds = jax.ShapeDtypeStruct((n_acc, HALF), jnp.float32)
    return pl.pallas_call(body, out_shape=(sds, sds))(x, w1, degp)


def _tc_mid(pA, pB, gA, gB, degp, b, w, n, n_acc):
    def body(pA_ref, pB_ref, gA_ref, gB_ref, degp_ref, b_ref, w_ref,
             oA_ref, oB_ref):
        dinv = _dinv_from_parts(degp_ref, n)
        sA = pA_ref[0, :n, :] + pA_ref[1, :n, :] + gA_ref[:n, :]
        sB = pB_ref[0, :n, :] + pB_ref[1, :n, :] + gB_ref[:n, :]
        s = jnp.concatenate([sA, sB], axis=1)
        h = jnp.maximum(dinv * s + b_ref[...], 0.0)
        g = dinv * jnp.dot(h, w_ref[...], preferred_element_type=jnp.float32)
        oA_ref[:n, :] = g[:, :HALF]
        oB_ref[:n, :] = g[:, HALF:]

    sds = jax.ShapeDtypeStruct((n_acc, HALF), jnp.float32)
    return pl.pallas_call(body, out_shape=(sds, sds))(
        pA, pB, gA, gB, degp, b, w)


def _tc_out(pA, pB, gA, gB, degp, b, w, bfc, n):
    def body(pA_ref, pB_ref, gA_ref, gB_ref, degp_ref, b_ref, w_ref,
             bfc_ref, o_ref):
        dinv = _dinv_from_parts(degp_ref, n)
        sA = pA_ref[0, :n, :] + pA_ref[1, :n, :] + gA_ref[:n, :]
        sB = pB_ref[0, :n, :] + pB_ref[1, :n, :] + gB_ref[:n, :]
        s = jnp.concatenate([sA, sB], axis=1)
        h = jnp.maximum(dinv * s + b_ref[...], 0.0)
        o_ref[...] = jnp.dot(h, w_ref[...],
                             preferred_element_type=jnp.float32) + bfc_ref[...]

    return pl.pallas_call(
        body, out_shape=jax.ShapeDtypeStruct((n, w.shape[1]), jnp.float32),
    )(pA, pB, gA, gB, degp, b, w, bfc)


def kernel(x, edge_index, W1, b1, W2, b2, Wfc, bfc):
    n, d_in = x.shape
    e = edge_index.shape[1]

    n_acc = -(-(n + 1) // (NS * 8)) * (NS * 8)
    grain = NC * NS * CHUNK * NBUF
    e_pad = -(-e // grain) * grain

    src = edge_index[0].astype(jnp.int32)
    dst = edge_index[1].astype(jnp.int32)
    src2d = jnp.concatenate(
        [src, jnp.zeros((e_pad - e,), jnp.int32)]).reshape(-1, CHUNK)
    dst2d = jnp.concatenate(
        [dst, jnp.full((e_pad - e,), n, jnp.int32)]).reshape(-1, CHUNK)

    degp = _sc_degree(dst2d, n_acc, e_pad).reshape(NC, n_acc, LANES)

    g1A, g1B = _tc_in(x, W1, degp, n, n_acc)
    p1A, p1B = _sc_message(g1A, g1B, src2d, dst2d, n_acc, e_pad)
    g2A, g2B = _tc_mid(p1A.reshape(NC, n_acc, HALF), p1B.reshape(NC, n_acc, HALF),
                       g1A, g1B, degp, b1.reshape(1, -1), W2, n, n_acc)
    p2A, p2B = _sc_message(g2A, g2B, src2d, dst2d, n_acc, e_pad)
    out = _tc_out(p2A.reshape(NC, n_acc, HALF), p2B.reshape(NC, n_acc, HALF),
                  g2A, g2B, degp, b2.reshape(1, -1), Wfc, bfc.reshape(1, -1), n)
    return out
